# Initial kernel scaffold; baseline (speedup 1.0000x reference)
import jax
import jax.numpy as jnp
from jax.experimental import pallas as pl


def kernel(x, weight_x, weight_y, gaussian_kernel):
    raise NotImplementedError("write your pallas kernel here")


# speedup vs baseline: 41.6321x; 41.6321x over previous
import math

import jax
import jax.numpy as jnp
import numpy as np
from jax.experimental import pallas as pl

NBINS = 9
POOL = 8
GW = 16
UNFOLD = 14
H = W = 224
HP = WP = H // POOL


def _hog_body(x_ref, gk_ref, pool_ref, out_ref):
    xb0 = x_ref[0, 0].astype(jnp.bfloat16).astype(jnp.float32)
    xb1 = x_ref[0, 1].astype(jnp.bfloat16).astype(jnp.float32)
    xb2 = x_ref[0, 2].astype(jnp.bfloat16).astype(jnp.float32)
    xs = xb0 + xb1 + xb2

    xp = jnp.concatenate([xs[1:2], xs, xs[H - 2:H - 1]], axis=0)
    xp = jnp.concatenate([xp[:, 1:2], xp, xp[:, W - 2:W - 1]], axis=1)

    vs = xp[:-2] + 2.0 * xp[1:-1] + xp[2:]
    vd = xp[:-2] - xp[2:]
    gx = vs[:, :-2] - vs[:, 2:]
    gy = vd[:, :-2] + 2.0 * vd[:, 1:-1] + vd[:, 2:]

    norm = jnp.sqrt(gx * gx + gy * gy) * gk_ref[...]

    flip = (gx < 0.0) | ((gx == 0.0) & (gy < 0.0))
    fx = jnp.where(flip, -gx, gx)
    fy = jnp.where(flip, -gy, gy)
    binv = jnp.zeros(fx.shape, dtype=jnp.int32)
    for k in range(1, NBINS):
        beta = k * math.pi / NBINS
        t = fx * math.cos(beta) - fy * math.sin(beta)
        binv = binv + jnp.where(t >= 0.0, 1, 0).astype(jnp.int32)

    p = pool_ref[...]
    hist = []
    for k in range(NBINS):
        m = jnp.where(binv == k, norm, 0.0)
        pm = jax.lax.dot_general(p, m, (((1,), (0,)), ((), ())),
                                 preferred_element_type=jnp.float32,
                                 precision=jax.lax.Precision.HIGHEST)
        hist.append(jax.lax.dot_general(pm, p, (((1,), (1,)), ((), ())),
                                        preferred_element_type=jnp.float32,
                                        precision=jax.lax.Precision.HIGHEST))
    h3 = jnp.stack(hist, axis=0)

    n2 = jnp.sqrt(jnp.sum(h3 * h3, axis=0, keepdims=True))
    out_ref[0] = h3 / jnp.maximum(n2, 1e-12)


def kernel(x, weight_x, weight_y, gaussian_kernel):
    b = x.shape[0]
    rep = H // GW
    gk = jnp.tile(gaussian_kernel, (rep, rep))
    pool_mat = jnp.asarray(
        np.repeat(np.eye(HP, dtype=np.float32), POOL, axis=1))

    normed = pl.pallas_call(
        _hog_body,
        grid=(b,),
        in_specs=[
            pl.BlockSpec((1, 3, H, W), lambda i: (i, 0, 0, 0)),
            pl.BlockSpec((H, W), lambda i: (0, 0)),
            pl.BlockSpec((HP, H), lambda i: (0, 0)),
        ],
        out_specs=pl.BlockSpec((1, NBINS, HP, WP), lambda i: (i, 0, 0, 0)),
        out_shape=jax.ShapeDtypeStruct((b, NBINS, HP, WP), jnp.float32),
    )(x, gk, pool_mat)

    us = WP // UNFOLD
    feat = normed.transpose(0, 2, 3, 1)
    feat = feat.reshape(b, HP // us, us, WP // us, us, NBINS)
    feat = feat.transpose(0, 1, 3, 5, 2, 4)
    return feat.reshape(b, (HP // us) * (WP // us), NBINS * us * us)
